# baseline (speedup 1.0000x reference)
import functools

import jax
import jax.numpy as jnp
from jax import lax
from jax.experimental import pallas as pl
from jax.experimental.pallas import tpu as pltpu
from jax.experimental.pallas import tpu_sc as plsc

N = 10000
NPAD = 10240
ROWS_PER_SUB = NPAD // 16
E = 320000
CHUNK = 128
ECHUNKS = 2528
EPAD = ECHUNKS * CHUNK
CPW_DEG = ECHUNKS // 32
CPW_EDGE = ECHUNKS // 16
F = 128
FH = 64
NSET_E = 6
NSET_D = 4
DW = 32
NG = 64
NCLS = 10
EPS = 1e-5

_SC_MESH = plsc.VectorSubcoreMesh(core_axis_name="c", subcore_axis_name="s")
_SC_PARAMS = pltpu.CompilerParams(use_tc_tiling_on_sc=False)


@functools.partial(
    pl.kernel,
    out_type=jax.ShapeDtypeStruct((2, NPAD, DW), jnp.float32),
    mesh=_SC_MESH,
    compiler_params=_SC_PARAMS,
    scratch_types=(
        [pltpu.VMEM((CHUNK,), jnp.int32)] * NSET_D
        + [pltpu.VMEM((CHUNK, DW), jnp.float32)]
        + [pltpu.VMEM_SHARED((NPAD, DW), jnp.float32)]
        + [pltpu.SemaphoreType.DMA] * (2 * NSET_D)
    ),
)
def _sc_degree(ridx_hbm, ones_hbm, zeros_hbm, out_hbm,
               rb0, rb1, rb2, rb3, ones_v, shared,
               i0, i1, i2, i3, s0, s1, s2, s3):
    cid = lax.axis_index("c")
    sid = lax.axis_index("s")
    wid = cid * 16 + sid
    rb = (rb0, rb1, rb2, rb3)
    isem = (i0, i1, i2, i3)
    ssem = (s0, s1, s2, s3)

    def fetch_idx(j, t):
        pltpu.async_copy(ridx_hbm.at[wid, j], rb[t], isem[t])

    def wait_idx(t):
        pltpu.make_async_copy(ridx_hbm.at[0, 0], rb[t], isem[t]).wait()

    def scatter(t):
        pltpu.async_copy(ones_v, shared.at[rb[t]], ssem[t], add=True)

    def wait_scatter(t):
        pltpu.make_async_copy(ones_v, shared.at[rb[t]], ssem[t]).wait()

    for t in range(NSET_D):
        fetch_idx(t, t)
    pltpu.sync_copy(ones_hbm, ones_v)
    pltpu.sync_copy(zeros_hbm, shared.at[pl.ds(sid * ROWS_PER_SUB, ROWS_PER_SUB)])
    plsc.subcore_barrier()

    def body(i, carry):
        j0 = NSET_D * i
        for t in range(NSET_D):
            j = j0 + t

            @pl.when(j < CPW_DEG)
            def _():
                wait_idx(t)
                scatter(t)
        for t in range(NSET_D):
            jn = j0 + t + NSET_D

            @pl.when(jn < CPW_DEG)
            def _():
                wait_scatter(t)
                fetch_idx(jn, t)
        return carry

    lax.fori_loop(0, (CPW_DEG + NSET_D - 1) // NSET_D, body, 0)
    for t in range(NSET_D):
        wait_scatter((CPW_DEG - NSET_D + t) % NSET_D)
    plsc.subcore_barrier()
    sl = pl.ds(sid * ROWS_PER_SUB, ROWS_PER_SUB)
    pltpu.sync_copy(shared.at[sl], out_hbm.at[cid, sl])


@functools.partial(
    pl.kernel,
    out_type=jax.ShapeDtypeStruct((2, NPAD, FH), jnp.float32),
    mesh=_SC_MESH,
    compiler_params=_SC_PARAMS,
    scratch_types=(
        [pltpu.VMEM((CHUNK,), jnp.int32)] * NSET_E
        + [pltpu.VMEM((CHUNK,), jnp.int32)] * NSET_E
        + [pltpu.VMEM((CHUNK, FH), jnp.float32)] * NSET_E
        + [pltpu.VMEM_SHARED((NPAD, FH), jnp.float32)]
        + [pltpu.SemaphoreType.DMA] * (3 * NSET_E)
    ),
)
def _sc_edge_layer(table_hbm, ridx_hbm, cidx_hbm, out_hbm,
                   rb0, rb1, rb2, rb3, rb4, rb5,
                   cb0, cb1, cb2, cb3, cb4, cb5,
                   rows0, rows1, rows2, rows3, rows4, rows5,
                   shared,
                   i0, i1, i2, i3, i4, i5,
                   g0, g1, g2, g3, g4, g5,
                   s0, s1, s2, s3, s4, s5):
    cid = lax.axis_index("c")
    sid = lax.axis_index("s")
    rb = (rb0, rb1, rb2, rb3, rb4, rb5)
    cb = (cb0, cb1, cb2, cb3, cb4, cb5)
    rows = (rows0, rows1, rows2, rows3, rows4, rows5)
    isem = (i0, i1, i2, i3, i4, i5)
    gsem = (g0, g1, g2, g3, g4, g5)
    ssem = (s0, s1, s2, s3, s4, s5)

    def fetch_idx(j, t):
        pltpu.async_copy(ridx_hbm.at[sid, j], rb[t], isem[t])
        pltpu.async_copy(cidx_hbm.at[sid, j], cb[t], isem[t])

    def wait_idx(t):
        pltpu.make_async_copy(ridx_hbm.at[0, 0], rb[t], isem[t]).wait()
        pltpu.make_async_copy(cidx_hbm.at[0, 0], cb[t], isem[t]).wait()

    def gather(t):
        pltpu.async_copy(table_hbm.at[cid].at[rb[t]], rows[t], gsem[t])

    def wait_gather(t):
        pltpu.make_async_copy(table_hbm.at[0].at[rb[t]], rows[t], gsem[t]).wait()

    def scatter(t):
        pltpu.async_copy(rows[t], shared.at[cb[t]], ssem[t], add=True)

    def wait_scatter(t):
        pltpu.make_async_copy(rows[t], shared.at[cb[t]], ssem[t]).wait()

    for t in range(NSET_E):
        fetch_idx(t, t)
    slx = pl.ds(sid * ROWS_PER_SUB, ROWS_PER_SUB)
    pltpu.async_copy(table_hbm.at[cid].at[slx], shared.at[slx], gsem[0])
    for t in range(NSET_E):
        wait_idx(t)
    pltpu.make_async_copy(table_hbm.at[cid].at[slx], shared.at[slx], gsem[0]).wait()
    plsc.subcore_barrier()
    for t in range(NSET_E):
        gather(t)

    def body(i, carry):
        j0 = NSET_E * i
        for t in range(NSET_E):
            j = j0 + t

            @pl.when(j < CPW_EDGE)
            def _():
                wait_gather(t)
                scatter(t)
        for t in range(NSET_E):
            jn = j0 + t + NSET_E

            @pl.when(jn < CPW_EDGE)
            def _():
                wait_scatter(t)
                fetch_idx(jn, t)
                wait_idx(t)
                gather(t)
        return carry

    lax.fori_loop(0, (CPW_EDGE + NSET_E - 1) // NSET_E, body, 0)
    for t in range(NSET_E):
        wait_scatter((CPW_EDGE - NSET_E + t) % NSET_E)
    plsc.subcore_barrier()
    sl = pl.ds(sid * ROWS_PER_SUB, ROWS_PER_SUB)
    pltpu.sync_copy(shared.at[sl], out_hbm.at[cid, sl])


def _bn_cols(h, g, b):
    m = jnp.mean(h, axis=0, keepdims=True)
    v = jnp.mean((h - m) * (h - m), axis=0, keepdims=True)
    return g * (h - m) * lax.rsqrt(v + EPS) + b


def _write_split(out_ref, val):
    out_ref[0, :N] = val[:, :FH]
    out_ref[0, N:] = jnp.zeros((NPAD - N, FH), jnp.float32)
    out_ref[1, :N] = val[:, FH:]
    out_ref[1, N:] = jnp.zeros((NPAD - N, FH), jnp.float32)


def _cat_split(ref):
    return jnp.concatenate([ref[0, :N], ref[1, :N]], axis=1)


def _tc_front_a_body(x_ref, g1_ref, b1_ref, Wf_ref, g2_ref, b2_ref,
                     W0_ref, h2_ref):
    x = x_ref[...]
    xb = _bn_cols(x, g1_ref[...], b1_ref[...])
    h1 = jnp.maximum(jnp.dot(xb, Wf_ref[...], preferred_element_type=jnp.float32), 0.0)
    hb = _bn_cols(h1, g2_ref[...], b2_ref[...])
    h2_ref[...] = jnp.dot(hb, W0_ref[...], preferred_element_type=jnp.float32)


def _tc_front_a(x, g1, b1, Wf, g2, b2, W0):
    return pl.pallas_call(
        _tc_front_a_body,
        out_shape=jax.ShapeDtypeStruct((N, F), jnp.float32),
    )(x, g1, b1, Wf, g2, b2, W0)


def _tc_front_b_body(h2_ref, degp_ref, h2s_ref, dis_ref):
    deg = degp_ref[0][:, :1] + degp_ref[1][:, :1] + 1.0
    dis = lax.rsqrt(deg)
    _write_split(h2s_ref, dis[:N] * h2_ref[...])
    dis_ref[...] = jnp.broadcast_to(dis, (NPAD, 8))


def _tc_front_b(h2, degp):
    return pl.pallas_call(
        _tc_front_b_body,
        out_shape=[
            jax.ShapeDtypeStruct((2, NPAD, FH), jnp.float32),
            jax.ShapeDtypeStruct((NPAD, 8), jnp.float32),
        ],
    )(h2, degp)


def _tc_mid_body(aggp_ref, dis_ref, bprev_ref, g_ref, bt_ref, W_ref,
                 out_ref):
    dis = dis_ref[:N, :1]
    agg = _cat_split(aggp_ref)
    h = jnp.maximum(dis * agg + bprev_ref[...], 0.0)
    hb = _bn_cols(h, g_ref[...], bt_ref[...])
    h2 = jnp.dot(hb, W_ref[...], preferred_element_type=jnp.float32)
    _write_split(out_ref, dis * h2)


def _tc_mid(aggp, dis, bprev, g, bt, W):
    return pl.pallas_call(
        _tc_mid_body,
        out_shape=jax.ShapeDtypeStruct((2, NPAD, FH), jnp.float32),
    )(aggp, dis, bprev, g, bt, W)


def _tc_tail_body(aggp_ref, dis_ref, b2_ref, batch_ref,
                  gfc_ref, bfc_ref, Wfc_ref, bfcb_ref,
                  ghid_ref, bhid_ref, Wcls_ref, bcls_ref, out_ref):
    dis = dis_ref[:N, :1]
    agg = _cat_split(aggp_ref)
    h3 = jnp.maximum(dis * agg + b2_ref[...], 0.0)
    gids = lax.broadcasted_iota(jnp.int32, (NG, N), 0)
    onehot = (gids == batch_ref[...]).astype(jnp.float32)
    pooled = jnp.dot(onehot, h3, preferred_element_type=jnp.float32)
    hb = _bn_cols(pooled, gfc_ref[...], bfc_ref[...])
    t = jnp.maximum(
        jnp.dot(hb, Wfc_ref[...], preferred_element_type=jnp.float32)
        + bfcb_ref[...], 0.0)
    tb = _bn_cols(t, ghid_ref[...], bhid_ref[...])
    logits = jnp.dot(tb, Wcls_ref[...], preferred_element_type=jnp.float32) \
        + bcls_ref[...]
    mx = jnp.max(logits, axis=-1, keepdims=True)
    s = logits - mx
    out_ref[...] = s - jnp.log(jnp.sum(jnp.exp(s), axis=-1, keepdims=True))


def _tc_tail(aggp, dis, b2, batch, gfc, bfc, Wfc, bfcb, ghid, bhid,
             Wcls, bcls):
    return pl.pallas_call(
        _tc_tail_body,
        out_shape=jax.ShapeDtypeStruct((NG, NCLS), jnp.float32),
    )(aggp, dis, b2, batch, gfc, bfc, Wfc, bfcb, ghid, bhid, Wcls, bcls)


def kernel(x, edge_index, batch, bn_feat_g, bn_feat_b, W_feat, bnc_g0,
           bnc_b0, Wc0, bc0, bnc_g1, bnc_b1, Wc1, bc1, bnc_g2, bnc_b2, Wc2,
           bc2, bn_fc_g, bn_fc_b, W_fc, b_fc, bn_hid_g, bn_hid_b, W_cls,
           b_cls):
    f32 = jnp.float32
    padi = jnp.full((EPAD - E,), N, jnp.int32)
    rflat = jnp.concatenate([edge_index[0], padi])
    cflat = jnp.concatenate([edge_index[1], padi])
    ridx32 = rflat.reshape(32, CPW_DEG, CHUNK)
    ridx16 = rflat.reshape(16, CPW_EDGE, CHUNK)
    cidx16 = cflat.reshape(16, CPW_EDGE, CHUNK)
    onesd = jnp.ones((CHUNK, DW), f32)
    zerosd = jnp.zeros((ROWS_PER_SUB, DW), f32)

    row = lambda v: v.reshape(1, -1)

    degp = _sc_degree(ridx32, onesd, zerosd)
    h2 = _tc_front_a(x, row(bn_feat_g), row(bn_feat_b), W_feat,
                     row(bnc_g0), row(bnc_b0), Wc0)
    h2s, dis = _tc_front_b(h2, degp)
    aggp = _sc_edge_layer(h2s, ridx16, cidx16)
    h2s = _tc_mid(aggp, dis, row(bc0), row(bnc_g1), row(bnc_b1), Wc1)
    aggp = _sc_edge_layer(h2s, ridx16, cidx16)
    h2s = _tc_mid(aggp, dis, row(bc1), row(bnc_g2), row(bnc_b2), Wc2)
    aggp = _sc_edge_layer(h2s, ridx16, cidx16)
    return _tc_tail(aggp, dis, row(bc2), row(batch),
                    row(bn_fc_g), row(bn_fc_b), W_fc, row(b_fc),
                    row(bn_hid_g), row(bn_hid_b), W_cls, row(b_cls))


# speedup vs baseline: 1.2417x; 1.2417x over previous
import functools

import jax
import jax.numpy as jnp
from jax import lax
from jax.experimental import pallas as pl
from jax.experimental.pallas import tpu as pltpu
from jax.experimental.pallas import tpu_sc as plsc

N = 10000
NPAD = 10240
ROWS_PER_SUB = NPAD // 16
E = 320000
CHUNK = 128
ECHUNKS = 2528
EPAD = ECHUNKS * CHUNK
CPW_DEG = ECHUNKS // 32
CPW_EDGE = ECHUNKS // 16
F = 128
FH = 64
NSET_E = 4
NSET_D = 4
DW = 32
NG = 64
NCLS = 10
EPS = 1e-5

_SC_MESH = plsc.VectorSubcoreMesh(core_axis_name="c", subcore_axis_name="s")
_SC_PARAMS = pltpu.CompilerParams(use_tc_tiling_on_sc=False)


@functools.partial(
    pl.kernel,
    out_type=jax.ShapeDtypeStruct((2, NPAD, DW), jnp.float32),
    mesh=_SC_MESH,
    compiler_params=_SC_PARAMS,
    scratch_types=(
        [pltpu.VMEM((CHUNK,), jnp.int32)] * NSET_D
        + [pltpu.VMEM((CHUNK, DW), jnp.float32)]
        + [pltpu.VMEM_SHARED((NPAD, DW), jnp.float32)]
        + [pltpu.SemaphoreType.DMA] * (2 * NSET_D)
    ),
)
def _sc_degree(ridx_hbm, ones_hbm, zeros_hbm, out_hbm,
               rb0, rb1, rb2, rb3, ones_v, shared,
               i0, i1, i2, i3, s0, s1, s2, s3):
    cid = lax.axis_index("c")
    sid = lax.axis_index("s")
    wid = cid * 16 + sid
    rb = (rb0, rb1, rb2, rb3)
    isem = (i0, i1, i2, i3)
    ssem = (s0, s1, s2, s3)

    def fetch_idx(j, t):
        pltpu.async_copy(ridx_hbm.at[wid, j], rb[t], isem[t])

    def wait_idx(t):
        pltpu.make_async_copy(ridx_hbm.at[0, 0], rb[t], isem[t]).wait()

    def scatter(t):
        pltpu.async_copy(ones_v, shared.at[rb[t]], ssem[t], add=True)

    def wait_scatter(t):
        pltpu.make_async_copy(ones_v, shared.at[rb[t]], ssem[t]).wait()

    for t in range(NSET_D):
        fetch_idx(t, t)
    pltpu.sync_copy(ones_hbm, ones_v)
    pltpu.sync_copy(zeros_hbm, shared.at[pl.ds(sid * ROWS_PER_SUB, ROWS_PER_SUB)])
    plsc.subcore_barrier()

    def body(i, carry):
        j0 = NSET_D * i
        for t in range(NSET_D):
            j = j0 + t

            @pl.when(j < CPW_DEG)
            def _():
                wait_idx(t)
                scatter(t)
        for t in range(NSET_D):
            jn = j0 + t + NSET_D

            @pl.when(jn < CPW_DEG)
            def _():
                wait_scatter(t)
                fetch_idx(jn, t)
        return carry

    lax.fori_loop(0, (CPW_DEG + NSET_D - 1) // NSET_D, body, 0)
    for t in range(NSET_D):
        wait_scatter((CPW_DEG - NSET_D + t) % NSET_D)
    plsc.subcore_barrier()
    sl = pl.ds(sid * ROWS_PER_SUB, ROWS_PER_SUB)
    pltpu.sync_copy(shared.at[sl], out_hbm.at[cid, sl])


@functools.partial(
    pl.kernel,
    out_type=jax.ShapeDtypeStruct((2, NPAD, FH), jnp.float32),
    mesh=_SC_MESH,
    compiler_params=_SC_PARAMS,
    scratch_types=(
        [pltpu.VMEM((CHUNK,), jnp.int32)] * NSET_E
        + [pltpu.VMEM((CHUNK,), jnp.int32)] * NSET_E
        + [pltpu.VMEM((CHUNK, FH), jnp.float32)] * NSET_E
        + [pltpu.VMEM_SHARED((NPAD, FH), jnp.float32)]
        + [pltpu.VMEM_SHARED((NPAD, FH), jnp.float32)]
        + [pltpu.SemaphoreType.DMA] * (3 * NSET_E)
    ),
)
def _sc_edge_layer(table_hbm, ridx_hbm, cidx_hbm, out_hbm,
                   rb0, rb1, rb2, rb3,
                   cb0, cb1, cb2, cb3,
                   rows0, rows1, rows2, rows3,
                   shared, table_s,
                   i0, i1, i2, i3,
                   g0, g1, g2, g3,
                   s0, s1, s2, s3):
    cid = lax.axis_index("c")
    sid = lax.axis_index("s")
    rb = (rb0, rb1, rb2, rb3)
    cb = (cb0, cb1, cb2, cb3)
    rows = (rows0, rows1, rows2, rows3)
    isem = (i0, i1, i2, i3)
    gsem = (g0, g1, g2, g3)
    ssem = (s0, s1, s2, s3)

    def fetch_idx(j, t):
        pltpu.async_copy(ridx_hbm.at[sid, j], rb[t], isem[t])
        pltpu.async_copy(cidx_hbm.at[sid, j], cb[t], isem[t])

    def wait_idx(t):
        pltpu.make_async_copy(ridx_hbm.at[0, 0], rb[t], isem[t]).wait()
        pltpu.make_async_copy(cidx_hbm.at[0, 0], cb[t], isem[t]).wait()

    def gather(t):
        pltpu.async_copy(table_s.at[rb[t]], rows[t], gsem[t])

    def wait_gather(t):
        pltpu.make_async_copy(table_s.at[rb[t]], rows[t], gsem[t]).wait()

    def scatter(t):
        pltpu.async_copy(rows[t], shared.at[cb[t]], ssem[t], add=True)

    def wait_scatter(t):
        pltpu.make_async_copy(rows[t], shared.at[cb[t]], ssem[t]).wait()

    for t in range(NSET_E):
        fetch_idx(t, t)
    slx = pl.ds(sid * ROWS_PER_SUB, ROWS_PER_SUB)
    pltpu.async_copy(table_hbm.at[cid].at[slx], shared.at[slx], gsem[0])
    pltpu.async_copy(table_hbm.at[cid].at[slx], table_s.at[slx], gsem[1])
    for t in range(NSET_E):
        wait_idx(t)
    pltpu.make_async_copy(table_hbm.at[cid].at[slx], shared.at[slx], gsem[0]).wait()
    pltpu.make_async_copy(table_hbm.at[cid].at[slx], table_s.at[slx], gsem[1]).wait()
    plsc.subcore_barrier()
    for t in range(NSET_E):
        gather(t)

    def body(i, carry):
        j0 = NSET_E * i
        for t in range(NSET_E):
            j = j0 + t

            @pl.when(j < CPW_EDGE)
            def _():
                wait_gather(t)
                scatter(t)
        for t in range(NSET_E):
            jn = j0 + t + NSET_E

            @pl.when(jn < CPW_EDGE)
            def _():
                wait_scatter(t)
                fetch_idx(jn, t)
                wait_idx(t)
                gather(t)
        return carry

    lax.fori_loop(0, (CPW_EDGE + NSET_E - 1) // NSET_E, body, 0)
    for t in range(NSET_E):
        wait_scatter((CPW_EDGE - NSET_E + t) % NSET_E)
    plsc.subcore_barrier()
    sl = pl.ds(sid * ROWS_PER_SUB, ROWS_PER_SUB)
    pltpu.sync_copy(shared.at[sl], out_hbm.at[cid, sl])


def _bn_cols(h, g, b):
    m = jnp.mean(h, axis=0, keepdims=True)
    v = jnp.mean((h - m) * (h - m), axis=0, keepdims=True)
    return g * (h - m) * lax.rsqrt(v + EPS) + b


def _write_split(out_ref, val):
    out_ref[0, :N] = val[:, :FH]
    out_ref[0, N:] = jnp.zeros((NPAD - N, FH), jnp.float32)
    out_ref[1, :N] = val[:, FH:]
    out_ref[1, N:] = jnp.zeros((NPAD - N, FH), jnp.float32)


def _cat_split(ref):
    return jnp.concatenate([ref[0, :N], ref[1, :N]], axis=1)


def _tc_front_a_body(x_ref, g1_ref, b1_ref, Wf_ref, g2_ref, b2_ref,
                     W0_ref, h2_ref):
    x = x_ref[...]
    xb = _bn_cols(x, g1_ref[...], b1_ref[...])
    h1 = jnp.maximum(jnp.dot(xb, Wf_ref[...], preferred_element_type=jnp.float32), 0.0)
    hb = _bn_cols(h1, g2_ref[...], b2_ref[...])
    h2_ref[...] = jnp.dot(hb, W0_ref[...], preferred_element_type=jnp.float32)


def _tc_front_a(x, g1, b1, Wf, g2, b2, W0):
    return pl.pallas_call(
        _tc_front_a_body,
        out_shape=jax.ShapeDtypeStruct((N, F), jnp.float32),
    )(x, g1, b1, Wf, g2, b2, W0)


def _tc_front_b_body(h2_ref, degp_ref, h2s_ref, dis_ref):
    deg = degp_ref[0][:, :1] + degp_ref[1][:, :1] + 1.0
    dis = lax.rsqrt(deg)
    _write_split(h2s_ref, dis[:N] * h2_ref[...])
    dis_ref[...] = jnp.broadcast_to(dis, (NPAD, 8))


def _tc_front_b(h2, degp):
    return pl.pallas_call(
        _tc_front_b_body,
        out_shape=[
            jax.ShapeDtypeStruct((2, NPAD, FH), jnp.float32),
            jax.ShapeDtypeStruct((NPAD, 8), jnp.float32),
        ],
    )(h2, degp)


def _tc_mid_body(aggp_ref, dis_ref, bprev_ref, g_ref, bt_ref, W_ref,
                 out_ref):
    dis = dis_ref[:N, :1]
    agg = _cat_split(aggp_ref)
    h = jnp.maximum(dis * agg + bprev_ref[...], 0.0)
    hb = _bn_cols(h, g_ref[...], bt_ref[...])
    h2 = jnp.dot(hb, W_ref[...], preferred_element_type=jnp.float32)
    _write_split(out_ref, dis * h2)


def _tc_mid(aggp, dis, bprev, g, bt, W):
    return pl.pallas_call(
        _tc_mid_body,
        out_shape=jax.ShapeDtypeStruct((2, NPAD, FH), jnp.float32),
    )(aggp, dis, bprev, g, bt, W)


def _tc_tail_body(aggp_ref, dis_ref, b2_ref, batch_ref,
                  gfc_ref, bfc_ref, Wfc_ref, bfcb_ref,
                  ghid_ref, bhid_ref, Wcls_ref, bcls_ref, out_ref):
    dis = dis_ref[:N, :1]
    agg = _cat_split(aggp_ref)
    h3 = jnp.maximum(dis * agg + b2_ref[...], 0.0)
    gids = lax.broadcasted_iota(jnp.int32, (NG, N), 0)
    onehot = (gids == batch_ref[...]).astype(jnp.float32)
    pooled = jnp.dot(onehot, h3, preferred_element_type=jnp.float32)
    hb = _bn_cols(pooled, gfc_ref[...], bfc_ref[...])
    t = jnp.maximum(
        jnp.dot(hb, Wfc_ref[...], preferred_element_type=jnp.float32)
        + bfcb_ref[...], 0.0)
    tb = _bn_cols(t, ghid_ref[...], bhid_ref[...])
    logits = jnp.dot(tb, Wcls_ref[...], preferred_element_type=jnp.float32) \
        + bcls_ref[...]
    mx = jnp.max(logits, axis=-1, keepdims=True)
    s = logits - mx
    out_ref[...] = s - jnp.log(jnp.sum(jnp.exp(s), axis=-1, keepdims=True))


def _tc_tail(aggp, dis, b2, batch, gfc, bfc, Wfc, bfcb, ghid, bhid,
             Wcls, bcls):
    return pl.pallas_call(
        _tc_tail_body,
        out_shape=jax.ShapeDtypeStruct((NG, NCLS), jnp.float32),
    )(aggp, dis, b2, batch, gfc, bfc, Wfc, bfcb, ghid, bhid, Wcls, bcls)


def kernel(x, edge_index, batch, bn_feat_g, bn_feat_b, W_feat, bnc_g0,
           bnc_b0, Wc0, bc0, bnc_g1, bnc_b1, Wc1, bc1, bnc_g2, bnc_b2, Wc2,
           bc2, bn_fc_g, bn_fc_b, W_fc, b_fc, bn_hid_g, bn_hid_b, W_cls,
           b_cls):
    f32 = jnp.float32
    padi = jnp.full((EPAD - E,), N, jnp.int32)
    rflat = jnp.concatenate([edge_index[0], padi])
    cflat = jnp.concatenate([edge_index[1], padi])
    ridx32 = rflat.reshape(32, CPW_DEG, CHUNK)
    ridx16 = rflat.reshape(16, CPW_EDGE, CHUNK)
    cidx16 = cflat.reshape(16, CPW_EDGE, CHUNK)
    onesd = jnp.ones((CHUNK, DW), f32)
    zerosd = jnp.zeros((ROWS_PER_SUB, DW), f32)

    row = lambda v: v.reshape(1, -1)

    degp = _sc_degree(ridx32, onesd, zerosd)
    h2 = _tc_front_a(x, row(bn_feat_g), row(bn_feat_b), W_feat,
                     row(bnc_g0), row(bnc_b0), Wc0)
    h2s, dis = _tc_front_b(h2, degp)
    aggp = _sc_edge_layer(h2s, ridx16, cidx16)
    h2s = _tc_mid(aggp, dis, row(bc0), row(bnc_g1), row(bnc_b1), Wc1)
    aggp = _sc_edge_layer(h2s, ridx16, cidx16)
    h2s = _tc_mid(aggp, dis, row(bc1), row(bnc_g2), row(bnc_b2), Wc2)
    aggp = _sc_edge_layer(h2s, ridx16, cidx16)
    return _tc_tail(aggp, dis, row(bc2), row(batch),
                    row(bn_fc_g), row(bn_fc_b), W_fc, row(b_fc),
                    row(bn_hid_g), row(bn_hid_b), W_cls, row(b_cls))


# speedup vs baseline: 1.2602x; 1.0149x over previous
import functools

import jax
import jax.numpy as jnp
from jax import lax
from jax.experimental import pallas as pl
from jax.experimental.pallas import tpu as pltpu
from jax.experimental.pallas import tpu_sc as plsc

N = 10000
NPAD = 10240
ROWS_PER_SUB = NPAD // 16
E = 320000
CHUNK = 128
ECHUNKS = 2528
EPAD = ECHUNKS * CHUNK
CPW_DEG = ECHUNKS // 32
ECHUNK = 80
CPW_EDGE = E // (16 * ECHUNK)
F = 128
FH = 64
NSET_E = 8
NSET_D = 4
DW = 32
NG = 64
NCLS = 10
EPS = 1e-5

_SC_MESH = plsc.VectorSubcoreMesh(core_axis_name="c", subcore_axis_name="s")
_SC_PARAMS = pltpu.CompilerParams(use_tc_tiling_on_sc=False)


@functools.partial(
    pl.kernel,
    out_type=jax.ShapeDtypeStruct((2, NPAD, DW), jnp.float32),
    mesh=_SC_MESH,
    compiler_params=_SC_PARAMS,
    scratch_types=(
        [pltpu.VMEM((CHUNK,), jnp.int32)] * NSET_D
        + [pltpu.VMEM((CHUNK, DW), jnp.float32)]
        + [pltpu.VMEM_SHARED((NPAD, DW), jnp.float32)]
        + [pltpu.SemaphoreType.DMA] * (2 * NSET_D)
    ),
)
def _sc_degree(ridx_hbm, ones_hbm, zeros_hbm, out_hbm,
               rb0, rb1, rb2, rb3, ones_v, shared,
               i0, i1, i2, i3, s0, s1, s2, s3):
    cid = lax.axis_index("c")
    sid = lax.axis_index("s")
    wid = cid * 16 + sid
    rb = (rb0, rb1, rb2, rb3)
    isem = (i0, i1, i2, i3)
    ssem = (s0, s1, s2, s3)

    def fetch_idx(j, t):
        pltpu.async_copy(ridx_hbm.at[wid, j], rb[t], isem[t])

    def wait_idx(t):
        pltpu.make_async_copy(ridx_hbm.at[0, 0], rb[t], isem[t]).wait()

    def scatter(t):
        pltpu.async_copy(ones_v, shared.at[rb[t]], ssem[t], add=True)

    def wait_scatter(t):
        pltpu.make_async_copy(ones_v, shared.at[rb[t]], ssem[t]).wait()

    for t in range(NSET_D):
        fetch_idx(t, t)
    pltpu.sync_copy(ones_hbm, ones_v)
    pltpu.sync_copy(zeros_hbm, shared.at[pl.ds(sid * ROWS_PER_SUB, ROWS_PER_SUB)])
    plsc.subcore_barrier()

    def body(i, carry):
        j0 = NSET_D * i
        for t in range(NSET_D):
            j = j0 + t

            @pl.when(j < CPW_DEG)
            def _():
                wait_idx(t)
                scatter(t)
        for t in range(NSET_D):
            jn = j0 + t + NSET_D

            @pl.when(jn < CPW_DEG)
            def _():
                wait_scatter(t)
                fetch_idx(jn, t)
        return carry

    lax.fori_loop(0, (CPW_DEG + NSET_D - 1) // NSET_D, body, 0)
    for t in range(NSET_D):
        wait_scatter((CPW_DEG - NSET_D + t) % NSET_D)
    plsc.subcore_barrier()
    sl = pl.ds(sid * ROWS_PER_SUB, ROWS_PER_SUB)
    pltpu.sync_copy(shared.at[sl], out_hbm.at[cid, sl])


@functools.partial(
    pl.kernel,
    out_type=jax.ShapeDtypeStruct((2, NPAD, FH), jnp.float32),
    mesh=_SC_MESH,
    compiler_params=_SC_PARAMS,
    scratch_types=(
        [pltpu.VMEM((ECHUNK,), jnp.int32)] * NSET_E
        + [pltpu.VMEM((ECHUNK,), jnp.int32)] * NSET_E
        + [pltpu.VMEM((ECHUNK, FH), jnp.float32)] * NSET_E
        + [pltpu.VMEM_SHARED((NPAD, FH), jnp.float32)]
        + [pltpu.VMEM_SHARED((NPAD, FH), jnp.float32)]
        + [pltpu.SemaphoreType.DMA] * (3 * NSET_E)
    ),
)
def _sc_edge_layer(table_hbm, ridx_hbm, cidx_hbm, out_hbm,
                   rb0, rb1, rb2, rb3, rb4, rb5, rb6, rb7,
                   cb0, cb1, cb2, cb3, cb4, cb5, cb6, cb7,
                   rows0, rows1, rows2, rows3, rows4, rows5, rows6, rows7,
                   shared, table_s,
                   i0, i1, i2, i3, i4, i5, i6, i7,
                   g0, g1, g2, g3, g4, g5, g6, g7,
                   s0, s1, s2, s3, s4, s5, s6, s7):
    cid = lax.axis_index("c")
    sid = lax.axis_index("s")
    rb = (rb0, rb1, rb2, rb3, rb4, rb5, rb6, rb7)
    cb = (cb0, cb1, cb2, cb3, cb4, cb5, cb6, cb7)
    rows = (rows0, rows1, rows2, rows3, rows4, rows5, rows6, rows7)
    isem = (i0, i1, i2, i3, i4, i5, i6, i7)
    gsem = (g0, g1, g2, g3, g4, g5, g6, g7)
    ssem = (s0, s1, s2, s3, s4, s5, s6, s7)

    def fetch_idx(j, t):
        pltpu.async_copy(ridx_hbm.at[sid, j], rb[t], isem[t])
        pltpu.async_copy(cidx_hbm.at[sid, j], cb[t], isem[t])

    def wait_idx(t):
        pltpu.make_async_copy(ridx_hbm.at[0, 0], rb[t], isem[t]).wait()
        pltpu.make_async_copy(cidx_hbm.at[0, 0], cb[t], isem[t]).wait()

    def gather(t):
        pltpu.async_copy(table_s.at[rb[t]], rows[t], gsem[t])

    def wait_gather(t):
        pltpu.make_async_copy(table_s.at[rb[t]], rows[t], gsem[t]).wait()

    def scatter(t):
        pltpu.async_copy(rows[t], shared.at[cb[t]], ssem[t], add=True)

    def wait_scatter(t):
        pltpu.make_async_copy(rows[t], shared.at[cb[t]], ssem[t]).wait()

    for t in range(NSET_E):
        fetch_idx(t, t)
    slx = pl.ds(sid * ROWS_PER_SUB, ROWS_PER_SUB)
    pltpu.async_copy(table_hbm.at[cid].at[slx], shared.at[slx], gsem[0])
    pltpu.async_copy(table_hbm.at[cid].at[slx], table_s.at[slx], gsem[1])
    for t in range(NSET_E):
        wait_idx(t)
    pltpu.make_async_copy(table_hbm.at[cid].at[slx], shared.at[slx], gsem[0]).wait()
    pltpu.make_async_copy(table_hbm.at[cid].at[slx], table_s.at[slx], gsem[1]).wait()
    plsc.subcore_barrier()
    for t in range(NSET_E):
        gather(t)

    def body(i, carry):
        j0 = NSET_E * i
        for t in range(NSET_E):
            j = j0 + t

            @pl.when(j < CPW_EDGE)
            def _():
                wait_gather(t)
                scatter(t)
        for t in range(NSET_E):
            jn = j0 + t + NSET_E

            @pl.when(jn < CPW_EDGE)
            def _():
                wait_scatter(t)
                fetch_idx(jn, t)
                wait_idx(t)
                gather(t)
        return carry

    lax.fori_loop(0, (CPW_EDGE + NSET_E - 1) // NSET_E, body, 0)
    for t in range(NSET_E):
        wait_scatter((CPW_EDGE - NSET_E + t) % NSET_E)
    plsc.subcore_barrier()
    sl = pl.ds(sid * ROWS_PER_SUB, ROWS_PER_SUB)
    pltpu.sync_copy(shared.at[sl], out_hbm.at[cid, sl])


def _bn_cols(h, g, b):
    m = jnp.mean(h, axis=0, keepdims=True)
    v = jnp.mean((h - m) * (h - m), axis=0, keepdims=True)
    return g * (h - m) * lax.rsqrt(v + EPS) + b


def _write_split(out_ref, val):
    out_ref[0, :N] = val[:, :FH]
    out_ref[0, N:] = jnp.zeros((NPAD - N, FH), jnp.float32)
    out_ref[1, :N] = val[:, FH:]
    out_ref[1, N:] = jnp.zeros((NPAD - N, FH), jnp.float32)


def _cat_split(ref):
    return jnp.concatenate([ref[0, :N], ref[1, :N]], axis=1)


def _tc_front_a_body(x_ref, g1_ref, b1_ref, Wf_ref, g2_ref, b2_ref,
                     W0_ref, h2_ref):
    x = x_ref[...]
    xb = _bn_cols(x, g1_ref[...], b1_ref[...])
    h1 = jnp.maximum(jnp.dot(xb, Wf_ref[...], preferred_element_type=jnp.float32), 0.0)
    hb = _bn_cols(h1, g2_ref[...], b2_ref[...])
    h2_ref[...] = jnp.dot(hb, W0_ref[...], preferred_element_type=jnp.float32)


def _tc_front_a(x, g1, b1, Wf, g2, b2, W0):
    return pl.pallas_call(
        _tc_front_a_body,
        out_shape=jax.ShapeDtypeStruct((N, F), jnp.float32),
    )(x, g1, b1, Wf, g2, b2, W0)


def _tc_front_b_body(h2_ref, degp_ref, h2s_ref, dis_ref):
    deg = degp_ref[0][:, :1] + degp_ref[1][:, :1] + 1.0
    dis = lax.rsqrt(deg)
    _write_split(h2s_ref, dis[:N] * h2_ref[...])
    dis_ref[...] = jnp.broadcast_to(dis, (NPAD, 8))


def _tc_front_b(h2, degp):
    return pl.pallas_call(
        _tc_front_b_body,
        out_shape=[
            jax.ShapeDtypeStruct((2, NPAD, FH), jnp.float32),
            jax.ShapeDtypeStruct((NPAD, 8), jnp.float32),
        ],
    )(h2, degp)


def _tc_mid_body(aggp_ref, dis_ref, bprev_ref, g_ref, bt_ref, W_ref,
                 out_ref):
    dis = dis_ref[:N, :1]
    agg = _cat_split(aggp_ref)
    h = jnp.maximum(dis * agg + bprev_ref[...], 0.0)
    hb = _bn_cols(h, g_ref[...], bt_ref[...])
    h2 = jnp.dot(hb, W_ref[...], preferred_element_type=jnp.float32)
    _write_split(out_ref, dis * h2)


def _tc_mid(aggp, dis, bprev, g, bt, W):
    return pl.pallas_call(
        _tc_mid_body,
        out_shape=jax.ShapeDtypeStruct((2, NPAD, FH), jnp.float32),
    )(aggp, dis, bprev, g, bt, W)


def _tc_tail_body(aggp_ref, dis_ref, b2_ref, batch_ref,
                  gfc_ref, bfc_ref, Wfc_ref, bfcb_ref,
                  ghid_ref, bhid_ref, Wcls_ref, bcls_ref, out_ref):
    dis = dis_ref[:N, :1]
    agg = _cat_split(aggp_ref)
    h3 = jnp.maximum(dis * agg + b2_ref[...], 0.0)
    gids = lax.broadcasted_iota(jnp.int32, (NG, N), 0)
    onehot = (gids == batch_ref[...]).astype(jnp.float32)
    pooled = jnp.dot(onehot, h3, preferred_element_type=jnp.float32)
    hb = _bn_cols(pooled, gfc_ref[...], bfc_ref[...])
    t = jnp.maximum(
        jnp.dot(hb, Wfc_ref[...], preferred_element_type=jnp.float32)
        + bfcb_ref[...], 0.0)
    tb = _bn_cols(t, ghid_ref[...], bhid_ref[...])
    logits = jnp.dot(tb, Wcls_ref[...], preferred_element_type=jnp.float32) \
        + bcls_ref[...]
    mx = jnp.max(logits, axis=-1, keepdims=True)
    s = logits - mx
    out_ref[...] = s - jnp.log(jnp.sum(jnp.exp(s), axis=-1, keepdims=True))


def _tc_tail(aggp, dis, b2, batch, gfc, bfc, Wfc, bfcb, ghid, bhid,
             Wcls, bcls):
    return pl.pallas_call(
        _tc_tail_body,
        out_shape=jax.ShapeDtypeStruct((NG, NCLS), jnp.float32),
    )(aggp, dis, b2, batch, gfc, bfc, Wfc, bfcb, ghid, bhid, Wcls, bcls)


def kernel(x, edge_index, batch, bn_feat_g, bn_feat_b, W_feat, bnc_g0,
           bnc_b0, Wc0, bc0, bnc_g1, bnc_b1, Wc1, bc1, bnc_g2, bnc_b2, Wc2,
           bc2, bn_fc_g, bn_fc_b, W_fc, b_fc, bn_hid_g, bn_hid_b, W_cls,
           b_cls):
    f32 = jnp.float32
    padi = jnp.full((EPAD - E,), N, jnp.int32)
    rflat = jnp.concatenate([edge_index[0], padi])
    cflat = jnp.concatenate([edge_index[1], padi])
    ridx32 = rflat.reshape(32, CPW_DEG, CHUNK)
    ridx16 = edge_index[0].reshape(16, CPW_EDGE, ECHUNK)
    cidx16 = edge_index[1].reshape(16, CPW_EDGE, ECHUNK)
    onesd = jnp.ones((CHUNK, DW), f32)
    zerosd = jnp.zeros((ROWS_PER_SUB, DW), f32)

    row = lambda v: v.reshape(1, -1)

    degp = _sc_degree(ridx32, onesd, zerosd)
    h2 = _tc_front_a(x, row(bn_feat_g), row(bn_feat_b), W_feat,
                     row(bnc_g0), row(bnc_b0), Wc0)
    h2s, dis = _tc_front_b(h2, degp)
    aggp = _sc_edge_layer(h2s, ridx16, cidx16)
    h2s = _tc_mid(aggp, dis, row(bc0), row(bnc_g1), row(bnc_b1), Wc1)
    aggp = _sc_edge_layer(h2s, ridx16, cidx16)
    h2s = _tc_mid(aggp, dis, row(bc1), row(bnc_g2), row(bnc_b2), Wc2)
    aggp = _sc_edge_layer(h2s, ridx16, cidx16)
    return _tc_tail(aggp, dis, row(bc2), row(batch),
                    row(bn_fc_g), row(bn_fc_b), W_fc, row(b_fc),
                    row(bn_hid_g), row(bn_hid_b), W_cls, row(b_cls))
